# baseline (speedup 1.0000x reference)
import functools

import jax
import jax.numpy as jnp
from jax import lax
from jax.experimental import pallas as pl
from jax.experimental.pallas import tpu as pltpu
from jax.experimental.pallas import tpu_sc as plsc

HIDDEN = 768
L = 16
NVEC = HIDDEN // L
NC, NS = 2, 16
NW = NC * NS
TOKENS = 4 * 2048
RPW = TOKENS // NW
C = 64
NCHUNK = RPW // C
SEQ = 2048
EPS = 1e-10


_GATHER_DNUMS = lax.GatherDimensionNumbers(
    offset_dims=(), collapsed_slice_dims=(0,), start_index_map=(0,))


def _shuffle(x, perm):
    return lax.gather(x, perm, _GATHER_DNUMS, slice_sizes=(1,),
                      mode=lax.GatherScatterMode.PROMISE_IN_BOUNDS)


def _bcast_sum(x):
    lanes = lax.iota(jnp.int32, L)
    for k in (1, 2, 4, 8):
        perm = lax.reshape(lanes ^ k, (L, 1))
        x = x + _shuffle(x, perm)
    return x


def _rsqrt_vec(v):
    i = lax.bitcast_convert_type(v, jnp.int32)
    i = jnp.int32(0x5F3759DF) - lax.shift_right_logical(i, 1)
    y = lax.bitcast_convert_type(i, jnp.float32)
    for _ in range(3):
        y = y * (1.5 - 0.5 * v * y * y)
    return y


_mesh = plsc.VectorSubcoreMesh(core_axis_name="c", subcore_axis_name="s")


@functools.partial(
    pl.kernel,
    out_type=jax.ShapeDtypeStruct((TOKENS, HIDDEN), jnp.float32),
    mesh=_mesh,
    scratch_types=[
        pltpu.VMEM((RPW,), jnp.int32),
        pltpu.VMEM((C, HIDDEN), jnp.float32),
        pltpu.VMEM((C, HIDDEN), jnp.float32),
        pltpu.VMEM((HIDDEN,), jnp.float32),
        pltpu.VMEM((HIDDEN,), jnp.float32),
        pltpu.VMEM((HIDDEN,), jnp.float32),
        pltpu.SemaphoreType.DMA,
    ],
)
def _emb_ln_kernel(ids_hbm, word_hbm, pos_hbm, type_hbm, gamma_hbm, beta_hbm,
                   out_hbm, idx_v, wbuf, pbuf, t_v, g_v, b_v, sem):
    wid = lax.axis_index("s") * NC + lax.axis_index("c")
    base = wid * RPW
    s_base = lax.rem(base, SEQ)

    pltpu.sync_copy(ids_hbm.at[pl.ds(base, RPW)], idx_v)
    pltpu.sync_copy(type_hbm, t_v)
    pltpu.sync_copy(gamma_hbm, g_v)
    pltpu.sync_copy(beta_hbm, b_v)

    for g in range(NCHUNK):
        pltpu.async_copy(
            word_hbm.at[idx_v.at[pl.ds(g * C, C)]], wbuf, sem).wait()
        pltpu.sync_copy(pos_hbm.at[pl.ds(s_base + g * C, C)], pbuf)

        def row_body(r, carry):
            acc_s = jnp.zeros((L,), jnp.float32)
            acc_q = jnp.zeros((L,), jnp.float32)
            for v in range(NVEC):
                sl = pl.ds(v * L, L)
                x = wbuf[r, sl] + pbuf[r, sl] + t_v[sl]
                wbuf[r, sl] = x
                acc_s = acc_s + x
                acc_q = acc_q + x * x
            mean_v = _bcast_sum(acc_s) * (1.0 / HIDDEN)
            var_v = _bcast_sum(acc_q) * (1.0 / HIDDEN) - mean_v * mean_v
            inv = _rsqrt_vec(var_v + EPS)
            for v in range(NVEC):
                sl = pl.ds(v * L, L)
                x = wbuf[r, sl]
                a = g_v[sl] * inv
                wbuf[r, sl] = (x - mean_v) * a + b_v[sl]
            return carry

        lax.fori_loop(0, C, row_body, jnp.int32(0))

        pltpu.sync_copy(wbuf, out_hbm.at[pl.ds(base + g * C, C)])


def kernel(input_ids, word_emb, pos_emb, type_emb, gamma, beta):
    ids = input_ids.reshape(-1).astype(jnp.int32)
    out = _emb_ln_kernel(ids, word_emb, pos_emb[:SEQ], type_emb[0],
                         gamma, beta)
    b, s = input_ids.shape
    return out.reshape(b, s, HIDDEN)


# speedup vs baseline: 1.4716x; 1.4716x over previous
import functools

import jax
import jax.numpy as jnp
from jax import lax
from jax.experimental import pallas as pl
from jax.experimental.pallas import tpu as pltpu
from jax.experimental.pallas import tpu_sc as plsc

HIDDEN = 768
L = 16
NVEC = HIDDEN // L
NC, NS = 2, 16
NW = NC * NS
TOKENS = 4 * 2048
RPW = TOKENS // NW
C = 16
NCHUNK = RPW // C
RB = 8
SEQ = 2048
EPS = 1e-10


_GATHER_DNUMS = lax.GatherDimensionNumbers(
    offset_dims=(), collapsed_slice_dims=(0,), start_index_map=(0,))


def _shuffle(x, perm):
    return lax.gather(x, perm, _GATHER_DNUMS, slice_sizes=(1,),
                      mode=lax.GatherScatterMode.PROMISE_IN_BOUNDS)


def _bcast_sum(x):
    lanes = lax.iota(jnp.int32, L)
    for k in (1, 2, 4, 8):
        perm = lax.reshape(lanes ^ k, (L, 1))
        x = x + _shuffle(x, perm)
    return x


def _rsqrt_vec(v):
    i = lax.bitcast_convert_type(v, jnp.int32)
    i = jnp.int32(0x5F3759DF) - lax.shift_right_logical(i, 1)
    y = lax.bitcast_convert_type(i, jnp.float32)
    for _ in range(3):
        y = y * (1.5 - 0.5 * v * y * y)
    return y


_mesh = plsc.VectorSubcoreMesh(core_axis_name="c", subcore_axis_name="s")


@functools.partial(
    pl.kernel,
    out_type=jax.ShapeDtypeStruct((TOKENS, HIDDEN), jnp.float32),
    mesh=_mesh,
    scratch_types=[
        pltpu.VMEM((RPW,), jnp.int32),
        pltpu.VMEM((2 * C, HIDDEN), jnp.float32),
        pltpu.VMEM((2 * C, HIDDEN), jnp.float32),
        pltpu.VMEM((2 * C, HIDDEN), jnp.float32),
        pltpu.VMEM((HIDDEN,), jnp.float32),
        pltpu.VMEM((HIDDEN,), jnp.float32),
        pltpu.VMEM((HIDDEN,), jnp.float32),
        pltpu.SemaphoreType.DMA,
        pltpu.SemaphoreType.DMA,
        pltpu.SemaphoreType.DMA,
        pltpu.SemaphoreType.DMA,
        pltpu.SemaphoreType.DMA,
        pltpu.SemaphoreType.DMA,
    ],
)
def _emb_ln_kernel(ids_hbm, word_hbm, pos_hbm, type_hbm, gamma_hbm, beta_hbm,
                   out_hbm, idx_v, wbuf, pbuf, obuf, t_v, g_v, b_v,
                   gsem0, gsem1, psem0, psem1, osem0, osem1):
    wid = lax.axis_index("s") * NC + lax.axis_index("c")
    base = wid * RPW
    s_base = lax.rem(base, SEQ)

    pltpu.sync_copy(ids_hbm.at[pl.ds(base, RPW)], idx_v)
    pltpu.sync_copy(type_hbm, t_v)
    pltpu.sync_copy(gamma_hbm, g_v)
    pltpu.sync_copy(beta_hbm, b_v)

    def start_in(g, gsem, psem):
        boff = lax.rem(g, 2) * C
        pltpu.async_copy(word_hbm.at[idx_v.at[pl.ds(g * C, C)]],
                         wbuf.at[pl.ds(boff, C)], gsem)
        pltpu.async_copy(pos_hbm.at[pl.ds(s_base + g * C, C)],
                         pbuf.at[pl.ds(boff, C)], psem)

    start_in(0, gsem0, psem0)
    start_in(1, gsem1, psem1)

    def wait_bytes(dst, sem):
        pltpu.make_async_copy(pos_hbm.at[pl.ds(0, C)], dst, sem).wait()

    def compute_block(boff):
        sacc = [jnp.zeros((L,), jnp.float32) for _ in range(RB)]
        qacc = [jnp.zeros((L,), jnp.float32) for _ in range(RB)]
        for v in range(NVEC):
            sl = pl.ds(v * L, L)
            tv = t_v[sl]
            for r in range(RB):
                x = wbuf[boff + r, sl] + pbuf[boff + r, sl] + tv
                wbuf[boff + r, sl] = x
                sacc[r] = sacc[r] + x
                qacc[r] = qacc[r] + x * x
        mean = [None] * RB
        inv = [None] * RB
        for r in range(RB):
            mean[r] = _bcast_sum(sacc[r]) * (1.0 / HIDDEN)
            var = _bcast_sum(qacc[r]) * (1.0 / HIDDEN) - mean[r] * mean[r]
            inv[r] = _rsqrt_vec(var + EPS)
        for v in range(NVEC):
            sl = pl.ds(v * L, L)
            gv = g_v[sl]
            bv = b_v[sl]
            for r in range(RB):
                x = wbuf[boff + r, sl]
                obuf[boff + r, sl] = (x - mean[r]) * inv[r] * gv + bv

    def chunk_body(g, carry):
        b = lax.rem(g, 2)
        boff = b * C
        even = b == 0

        @pl.when(even)
        def _():
            wait_bytes(wbuf.at[pl.ds(boff, C)], gsem0)
            wait_bytes(pbuf.at[pl.ds(boff, C)], psem0)

        @pl.when(jnp.logical_not(even))
        def _():
            wait_bytes(wbuf.at[pl.ds(boff, C)], gsem1)
            wait_bytes(pbuf.at[pl.ds(boff, C)], psem1)

        @pl.when(jnp.logical_and(g >= 2, even))
        def _():
            wait_bytes(obuf.at[pl.ds(boff, C)], osem0)

        @pl.when(jnp.logical_and(g >= 2, jnp.logical_not(even)))
        def _():
            wait_bytes(obuf.at[pl.ds(boff, C)], osem1)

        def block_body(blk, carry2):
            compute_block(boff + blk * RB)
            return carry2

        lax.fori_loop(0, C // RB, block_body, jnp.int32(0))

        @pl.when(even)
        def _():
            pltpu.async_copy(obuf.at[pl.ds(boff, C)],
                             out_hbm.at[pl.ds(base + g * C, C)], osem0)

            @pl.when(g + 2 < NCHUNK)
            def _():
                start_in(g + 2, gsem0, psem0)

        @pl.when(jnp.logical_not(even))
        def _():
            pltpu.async_copy(obuf.at[pl.ds(boff, C)],
                             out_hbm.at[pl.ds(base + g * C, C)], osem1)

            @pl.when(g + 2 < NCHUNK)
            def _():
                start_in(g + 2, gsem1, psem1)

        return carry

    lax.fori_loop(0, NCHUNK, chunk_body, jnp.int32(0))

    wait_bytes(obuf.at[pl.ds(0, C)], osem0)
    wait_bytes(obuf.at[pl.ds(C, C)], osem1)


def kernel(input_ids, word_emb, pos_emb, type_emb, gamma, beta):
    ids = input_ids.reshape(-1).astype(jnp.int32)
    out = _emb_ln_kernel(ids, word_emb, pos_emb[:SEQ], type_emb[0],
                         gamma, beta)
    b, s = input_ids.shape
    return out.reshape(b, s, HIDDEN)
